# baseline (speedup 1.0000x reference)
import functools

import jax
import jax.numpy as jnp
from jax import lax
from jax.experimental import pallas as pl
from jax.experimental.pallas import tpu as pltpu
from jax.experimental.pallas import tpu_sc as plsc

EMB = 64
ROWW = 32
GRP = 128
NUM_WORKERS = 32
NGRP_TILE = 64
CHUNK = 4
MM_BLK = 512


def _gather_pipeline(tabs, base):
    for (ih, th, oh, iv, bv, gs, ss) in tabs:
        pltpu.sync_copy(ih.at[pl.ds(base, NGRP_TILE)], iv)

    def chunk(c, carry):
        handles = []
        for (ih, th, oh, iv, bv, gs, ss) in tabs:
            @pl.when(c > 0)
            def _drain(oh=oh, bv=bv, ss=ss):
                for b in range(CHUNK):
                    pltpu.make_async_copy(bv.at[b], oh.at[b], ss).wait()
            hs = []
            for b in range(CHUNK):
                hs.append(pltpu.async_copy(
                    th.at[iv.at[c * CHUNK + b]], bv.at[b], gs))
            handles.append(hs)
        for (ih, th, oh, iv, bv, gs, ss), hs in zip(tabs, handles):
            for b in range(CHUNK):
                hs[b].wait()
                g = base + c * CHUNK + b
                pltpu.async_copy(bv.at[b], oh.at[g], ss)
        return carry

    lax.fori_loop(0, NGRP_TILE // CHUNK, chunk, 0)
    for (ih, th, oh, iv, bv, gs, ss) in tabs:
        for b in range(CHUNK):
            pltpu.make_async_copy(bv.at[b], oh.at[b], ss).wait()


def _sc_gather_node_body(s_idx, o_idx, node_tab, s_out, o_out,
                         idx_s, idx_o, buf_s, buf_o,
                         gsem_s, gsem_o, ssem_s, ssem_o):
    wid = lax.axis_index("s") * 2 + lax.axis_index("c")
    base = wid * NGRP_TILE
    tabs = ((s_idx, node_tab, s_out, idx_s, buf_s, gsem_s, ssem_s),
            (o_idx, node_tab, o_out, idx_o, buf_o, gsem_o, ssem_o))
    _gather_pipeline(tabs, base)


def _sc_gather_rel_body(p_idx, rel_tab, p_out,
                        idx_p, buf_p, gsem_p, ssem_p):
    wid = lax.axis_index("s") * 2 + lax.axis_index("c")
    base = wid * NGRP_TILE
    tabs = ((p_idx, rel_tab, p_out, idx_p, buf_p, gsem_p, ssem_p),)
    _gather_pipeline(tabs, base)


_MESH = plsc.VectorSubcoreMesh(core_axis_name="c", subcore_axis_name="s")
_IDX_T = pltpu.VMEM((NGRP_TILE, GRP), jnp.int32)
_BUF_T = pltpu.VMEM((CHUNK, GRP, ROWW), jnp.int32)


def _sc_gather_node(s_idx, o_idx, node_tab):
    ngrp = s_idx.shape[0]
    out_t = jax.ShapeDtypeStruct((ngrp, GRP, ROWW), jnp.int32)
    f = functools.partial(
        pl.kernel,
        mesh=_MESH,
        compiler_params=pltpu.CompilerParams(use_tc_tiling_on_sc=False),
        out_type=(out_t, out_t),
        scratch_types=[_IDX_T, _IDX_T, _BUF_T, _BUF_T]
        + [pltpu.SemaphoreType.DMA] * 4,
    )(_sc_gather_node_body)
    return f(s_idx, o_idx, node_tab)


def _sc_gather_rel(p_idx, rel_tab):
    ngrp = p_idx.shape[0]
    out_t = jax.ShapeDtypeStruct((ngrp, GRP, ROWW), jnp.int32)
    f = functools.partial(
        pl.kernel,
        mesh=_MESH,
        compiler_params=pltpu.CompilerParams(use_tc_tiling_on_sc=False),
        out_type=(out_t,),
        scratch_types=[_IDX_T, _BUF_T] + [pltpu.SemaphoreType.DMA] * 2,
    )(_sc_gather_rel_body)
    return f(p_idx, rel_tab)[0]


def _mm_body(s_ref, p_ref, o_ref, we_s, wo_s, we_p, wo_p, we_o, wo_o,
             b_ref, out_ref):
    acc = b_ref[...]
    for ref, we, wo in ((s_ref, we_s, wo_s), (p_ref, we_p, wo_p),
                        (o_ref, we_o, wo_o)):
        w = ref[...]
        qe = lax.bitcast_convert_type(
            lax.shift_left(w, jnp.int32(16)), jnp.float32)
        qo = lax.bitcast_convert_type(
            lax.bitwise_and(w, jnp.int32(-65536)), jnp.float32)
        acc = acc + jnp.dot(qe, we[...], preferred_element_type=jnp.float32)
        acc = acc + jnp.dot(qo, wo[...], preferred_element_type=jnp.float32)
    out_ref[...] = acc


def _mm(s4, p4, o4, wparts, bb):
    n4 = s4.shape[0]
    grid = (n4 // MM_BLK,)
    in_spec = pl.BlockSpec((MM_BLK, 128), lambda i: (i, 0))
    w_spec = pl.BlockSpec((128, 256), lambda i: (0, 0))
    return pl.pallas_call(
        _mm_body,
        grid=grid,
        in_specs=[in_spec, in_spec, in_spec] + [w_spec] * 6
        + [pl.BlockSpec((1, 256), lambda i: (0, 0))],
        out_specs=pl.BlockSpec((MM_BLK, 256), lambda i: (i, 0)),
        out_shape=jax.ShapeDtypeStruct((n4, 256), jnp.float32),
    )(s4, p4, o4, *wparts, bb)


def _blockdiag4(wt):
    z = jnp.zeros((32, 64), jnp.float32)
    rows = []
    for g in range(4):
        blocks = [z] * 4
        blocks[g] = wt
        rows.append(jnp.concatenate(blocks, axis=1))
    return jnp.concatenate(rows, axis=0)


def _pack_table(tab):
    n = tab.shape[0]
    tb = tab.astype(jnp.bfloat16).reshape(n, ROWW, 2)
    return jax.lax.bitcast_convert_type(tb, jnp.int32)


def kernel(triples, node_table, relation_table, W, b):
    bsz, esz, _ = triples.shape
    n = bsz * esz
    t = triples.reshape(n, 3).astype(jnp.int32)
    s_idx = t[:, 0].reshape(n // GRP, GRP)
    p_idx = t[:, 1].reshape(n // GRP, GRP)
    o_idx = t[:, 2].reshape(n // GRP, GRP)

    nt_i32 = _pack_table(node_table)
    rt_i32 = _pack_table(relation_table)

    s3, o3 = _sc_gather_node(s_idx, o_idx, nt_i32)
    p3 = _sc_gather_rel(p_idx, rt_i32)
    s4 = s3.reshape(n // 4, 128)
    p4 = p3.reshape(n // 4, 128)
    o4 = o3.reshape(n // 4, 128)

    wparts = []
    for ti in range(3):
        wt = W[ti * EMB:(ti + 1) * EMB]
        wparts.append(_blockdiag4(wt[0::2]))
        wparts.append(_blockdiag4(wt[1::2]))
    bb = jnp.tile(b, 4).reshape(1, 256)

    out4 = _mm(s4, p4, o4, wparts, bb)
    return out4.reshape(bsz, esz, EMB)
